# Initial kernel scaffold; baseline (speedup 1.0000x reference)
import jax
import jax.numpy as jnp
from jax.experimental import pallas as pl


def kernel(x, token_table, pos_table):
    raise NotImplementedError("write your pallas kernel here")


# speedup vs baseline: 3.5151x; 3.5151x over previous
import functools

import jax
import jax.numpy as jnp
from jax import lax
from jax.experimental import pallas as pl
from jax.experimental.pallas import tpu as pltpu
from jax.experimental.pallas import tpu_sc as plsc

LANES = 16


def _make_sc_kernel(n_flat, vocab, maxlen, embed, n_workers, chunk):
    n_chunks = n_flat // (n_workers * chunk)
    b_per_w = n_flat // n_workers
    groups = chunk // maxlen
    mesh = plsc.VectorSubcoreMesh(core_axis_name="c", subcore_axis_name="s")

    @functools.partial(
        pl.kernel,
        mesh=mesh,
        out_type=jax.ShapeDtypeStruct((n_flat, embed), jnp.float32),
        scratch_types=[
            pltpu.VMEM((chunk,), jnp.int32),
            pltpu.VMEM((chunk, embed), jnp.float32),
            pltpu.VMEM((maxlen, embed), jnp.float32),
            pltpu.SemaphoreType.DMA,
        ],
        compiler_params=pltpu.CompilerParams(use_tc_tiling_on_sc=False),
    )
    def sc_kernel(x_hbm, tok_hbm, pos_hbm, out_hbm, idx_v, rows_v, pos_v, sem):
        wid = lax.axis_index("s") * 2 + lax.axis_index("c")
        base = wid * b_per_w
        pltpu.sync_copy(pos_hbm, pos_v)

        def chunk_body(g, carry):
            off = base + g * chunk
            pltpu.sync_copy(x_hbm.at[pl.ds(off, chunk)], idx_v)
            pltpu.async_copy(tok_hbm.at[idx_v], rows_v, sem).wait()

            def seq_body(r, carry2):
                for t in range(groups):
                    for k in range(embed // LANES):
                        plsc.addupdate(
                            rows_v.at[t * maxlen + r, pl.ds(k * LANES, LANES)],
                            pos_v[r, pl.ds(k * LANES, LANES)],
                        )
                return carry2

            lax.fori_loop(0, maxlen, seq_body, 0)
            pltpu.sync_copy(rows_v, out_hbm.at[pl.ds(off, chunk)])
            return carry

        lax.fori_loop(0, n_chunks, chunk_body, 0)

    return sc_kernel


def kernel(x, token_table, pos_table):
    batch, maxlen = x.shape
    vocab, embed = token_table.shape
    n_flat = batch * maxlen
    n_workers = 32
    chunk = 4 * maxlen
    assert n_flat % (n_workers * chunk) == 0

    x_flat = x.reshape(n_flat).astype(jnp.int32)
    sc = _make_sc_kernel(n_flat, vocab, maxlen, embed, n_workers, chunk)
    out = sc(x_flat, token_table, pos_table)
    return out.reshape(batch, maxlen, embed)


# speedup vs baseline: 4.1638x; 1.1845x over previous
import functools

import jax
import jax.numpy as jnp
from jax import lax
from jax.experimental import pallas as pl
from jax.experimental.pallas import tpu as pltpu
from jax.experimental.pallas import tpu_sc as plsc

LANES = 16
NBUF = 2


def _make_sc_kernel(n_flat, vocab, maxlen, embed, n_workers, chunk):
    n_chunks = n_flat // (n_workers * chunk)
    b_per_w = n_flat // n_workers
    groups = chunk // maxlen
    n_outer = n_chunks // NBUF
    mesh = plsc.VectorSubcoreMesh(core_axis_name="c", subcore_axis_name="s")

    @functools.partial(
        pl.kernel,
        mesh=mesh,
        out_type=jax.ShapeDtypeStruct((n_flat, embed), jnp.float32),
        scratch_types=[
            [pltpu.VMEM((chunk,), jnp.int32) for _ in range(NBUF)],
            [pltpu.VMEM((chunk, embed), jnp.float32) for _ in range(NBUF)],
            pltpu.VMEM((maxlen, embed), jnp.float32),
            [pltpu.SemaphoreType.DMA for _ in range(NBUF)],
            [pltpu.SemaphoreType.DMA for _ in range(NBUF)],
        ],
        compiler_params=pltpu.CompilerParams(use_tc_tiling_on_sc=False),
    )
    def sc_kernel(x_hbm, tok_hbm, pos_hbm, out_hbm, idx_v, rows_v, pos_v,
                  sem_g, sem_s):
        wid = lax.axis_index("s") * 2 + lax.axis_index("c")
        base = wid * b_per_w
        pltpu.sync_copy(pos_hbm, pos_v)

        def start_gather(b, off):
            pltpu.sync_copy(x_hbm.at[pl.ds(off, chunk)], idx_v[b])
            pltpu.async_copy(tok_hbm.at[idx_v[b]], rows_v[b], sem_g[b])

        def wait_gather(b):
            pltpu.make_async_copy(tok_hbm.at[idx_v[b]], rows_v[b],
                                  sem_g[b]).wait()

        def start_store(b, off):
            pltpu.async_copy(rows_v[b], out_hbm.at[pl.ds(off, chunk)],
                             sem_s[b])

        def wait_store(b, off):
            pltpu.make_async_copy(rows_v[b], out_hbm.at[pl.ds(off, chunk)],
                                  sem_s[b]).wait()

        def add_pos(b):
            def seq_body(r, carry):
                for k in range(embed // LANES):
                    v = pos_v[r, pl.ds(k * LANES, LANES)]
                    for t in range(groups):
                        plsc.addupdate(
                            rows_v[b].at[t * maxlen + r,
                                         pl.ds(k * LANES, LANES)], v)
                return carry

            lax.fori_loop(0, maxlen, seq_body, 0)

        for b in range(NBUF):
            start_gather(b, base + b * chunk)

        def outer(i, carry):
            g0 = i * NBUF
            for b in range(NBUF):
                off = base + (g0 + b) * chunk
                wait_gather(b)
                add_pos(b)
                start_store(b, off)
            for b in range(NBUF):
                off = base + (g0 + b) * chunk
                off2 = off + NBUF * chunk

                @pl.when(g0 + b + NBUF < n_chunks)
                def _():
                    wait_store(b, off)
                    start_gather(b, off2)

            return carry

        lax.fori_loop(0, n_outer, outer, 0)
        for b in range(NBUF):
            wait_store(b, base + (n_chunks - NBUF + b) * chunk)

    return sc_kernel


def kernel(x, token_table, pos_table):
    batch, maxlen = x.shape
    vocab, embed = token_table.shape
    n_flat = batch * maxlen
    n_workers = 32
    chunk = 4 * maxlen
    assert n_flat % (n_workers * chunk * NBUF) == 0

    x_flat = x.reshape(n_flat).astype(jnp.int32)
    sc = _make_sc_kernel(n_flat, vocab, maxlen, embed, n_workers, chunk)
    out = sc(x_flat, token_table, pos_table)
    return out.reshape(batch, maxlen, embed)
